# Initial kernel scaffold; baseline (speedup 1.0000x reference)
import jax
import jax.numpy as jnp
from jax.experimental import pallas as pl


def kernel(embeddings, indices):
    raise NotImplementedError("write your pallas kernel here")


# speedup vs baseline: 1.8437x; 1.8437x over previous
import functools

import jax
import jax.numpy as jnp
from jax import lax
from jax.experimental import pallas as pl
from jax.experimental.pallas import tpu as pltpu
from jax.experimental.pallas import tpu_sc as plsc

EMB_DIM = 64
NUM_WORKERS = 32
CHUNK = 1024


def _gather_body(n_chunks, rows_per_worker, table_hbm, idx_hbm, out_hbm,
                 idx_v, rows_v, sem):
    cid = lax.axis_index("c")
    sid = lax.axis_index("s")
    wid = sid * 2 + cid
    base = wid * rows_per_worker

    def body(g, carry):
        off = base + g * CHUNK
        pltpu.sync_copy(idx_hbm.at[pl.ds(off, CHUNK)], idx_v)
        pltpu.async_copy(table_hbm.at[idx_v], rows_v, sem).wait()
        pltpu.sync_copy(rows_v, out_hbm.at[pl.ds(off, CHUNK)])
        return carry

    lax.fori_loop(0, n_chunks, body, 0, unroll=False)


def kernel(embeddings, indices):
    flat_idx = indices.reshape(-1).astype(jnp.int32)
    total = flat_idx.shape[0]
    rows_per_worker = total // NUM_WORKERS
    n_chunks = rows_per_worker // CHUNK
    assert rows_per_worker * NUM_WORKERS == total
    assert n_chunks * CHUNK == rows_per_worker

    mesh = plsc.VectorSubcoreMesh(core_axis_name="c", subcore_axis_name="s")
    grab = pl.kernel(
        functools.partial(_gather_body, n_chunks, rows_per_worker),
        out_type=jax.ShapeDtypeStruct((total, EMB_DIM), jnp.float32),
        mesh=mesh,
        scratch_types=[
            pltpu.VMEM((CHUNK,), jnp.int32),
            pltpu.VMEM((CHUNK, EMB_DIM), jnp.float32),
            pltpu.SemaphoreType.DMA,
        ],
        compiler_params=pltpu.CompilerParams(use_tc_tiling_on_sc=False),
    )
    out = grab(embeddings, flat_idx)
    return out.reshape(indices.shape + (EMB_DIM,))


# speedup vs baseline: 1.8519x; 1.0044x over previous
import functools

import jax
import jax.numpy as jnp
from jax import lax
from jax.experimental import pallas as pl
from jax.experimental.pallas import tpu as pltpu
from jax.experimental.pallas import tpu_sc as plsc

EMB_DIM = 64
NUM_WORKERS = 32
CHUNK = 800


def _gather_body(n_chunks, rows_per_worker, table_hbm, idx_hbm, out_hbm,
                 idx_all, rows0, rows1, sg0, sg1, so0, so1):
    cid = lax.axis_index("c")
    sid = lax.axis_index("s")
    wid = sid * 2 + cid
    base = wid * rows_per_worker

    rows = (rows0, rows1)
    sem_g = (sg0, sg1)
    sem_o = (so0, so1)

    pltpu.sync_copy(idx_hbm.at[pl.ds(base, rows_per_worker)], idx_all)

    def idx_slice(g):
        return idx_all.at[pl.ds(g * CHUNK, CHUNK)]

    def fire_gather(g, j):
        pltpu.async_copy(table_hbm.at[idx_slice(g)], rows[j], sem_g[j])

    def wait_gather(j):
        pltpu.make_async_copy(table_hbm.at[idx_slice(0)], rows[j],
                              sem_g[j]).wait()

    def fire_out(g, j):
        pltpu.async_copy(rows[j], out_hbm.at[pl.ds(base + g * CHUNK, CHUNK)],
                         sem_o[j])

    def wait_out(j):
        pltpu.make_async_copy(rows[j], out_hbm.at[pl.ds(base, CHUNK)],
                              sem_o[j]).wait()

    fire_gather(0, 0)

    def pair(i, carry):
        for j in (0, 1):
            g = 2 * i + j

            @pl.when(g > 0)
            def _():
                wait_out(1 - j)

            @pl.when(g < n_chunks - 1)
            def _():
                fire_gather(g + 1, 1 - j)

            wait_gather(j)
            fire_out(g, j)
        return carry

    lax.fori_loop(0, n_chunks // 2, pair, 0, unroll=False)
    wait_out((n_chunks - 1) % 2)


def kernel(embeddings, indices):
    flat_idx = indices.reshape(-1).astype(jnp.int32)
    total = flat_idx.shape[0]
    rows_per_worker = total // NUM_WORKERS
    n_chunks = rows_per_worker // CHUNK
    assert rows_per_worker * NUM_WORKERS == total
    assert n_chunks * CHUNK == rows_per_worker and n_chunks % 2 == 0

    mesh = plsc.VectorSubcoreMesh(core_axis_name="c", subcore_axis_name="s")
    grab = pl.kernel(
        functools.partial(_gather_body, n_chunks, rows_per_worker),
        out_type=jax.ShapeDtypeStruct((total, EMB_DIM), jnp.float32),
        mesh=mesh,
        scratch_types=[
            pltpu.VMEM((rows_per_worker,), jnp.int32),
            pltpu.VMEM((CHUNK, EMB_DIM), jnp.float32),
            pltpu.VMEM((CHUNK, EMB_DIM), jnp.float32),
            pltpu.SemaphoreType.DMA,
            pltpu.SemaphoreType.DMA,
            pltpu.SemaphoreType.DMA,
            pltpu.SemaphoreType.DMA,
        ],
        compiler_params=pltpu.CompilerParams(use_tc_tiling_on_sc=False),
    )
    out = grab(embeddings, flat_idx)
    return out.reshape(indices.shape + (EMB_DIM,))
